# baseline (speedup 1.0000x reference)
import functools

import jax
import jax.numpy as jnp
from jax import lax
from jax.experimental import pallas as pl
from jax.experimental.pallas import tpu as pltpu
from jax.experimental.pallas import tpu_sc as plsc

N = 10000
E = 320000
H = 128

NC, NS = 2, 16
NW = NC * NS
CH = 80
SS = 5
S = CH * SS
PER_W = E // NW
SUP = PER_W // S

_MESH = plsc.VectorSubcoreMesh(
    core_axis_name="c", subcore_axis_name="s", num_cores=NC, num_subcores=NS)

_f32 = jnp.float32
_bf16 = jnp.bfloat16
_dot = functools.partial(jnp.dot, preferred_element_type=jnp.float32)


def _bdot(x, w):
    return jnp.dot(x.astype(_bf16), w.astype(_bf16),
                   preferred_element_type=jnp.float32)


def _elu(h):
    return jnp.where(h > 0, h, jnp.exp(jnp.minimum(h, 0.0)) - 1.0)


def _ln(h, g, beta):
    mu = jnp.mean(h, axis=-1, keepdims=True)
    var = jnp.mean((h - mu) ** 2, axis=-1, keepdims=True)
    return (h - mu) * lax.rsqrt(var + 1e-5) * g + beta


def _nrm2_body(pos_hbm, src_hbm, dst_hbm, nrm2_hbm, pos_v, isv, idv, nv):
    wid = lax.axis_index("s") * NC + lax.axis_index("c")
    base = wid * PER_W
    pltpu.sync_copy(pos_hbm, pos_v)

    def step(t, carry):
        off = base + t * S
        pltpu.sync_copy(src_hbm.at[pl.ds(off, S)], isv)
        pltpu.sync_copy(dst_hbm.at[pl.ds(off, S)], idv)
        for k in range(S // 16):
            sl = pl.ds(k * 16, 16)
            rs = isv[sl] * 3
            rd = idv[sl] * 3
            acc = jnp.zeros((16,), _f32)
            for comp in range(3):
                ps = plsc.load_gather(pos_v, [rs + comp])
                pd_ = plsc.load_gather(pos_v, [rd + comp])
                d = ps - pd_
                acc = acc + d * d
            nv[sl] = acc
        pltpu.sync_copy(nv, nrm2_hbm.at[pl.ds(off, S)])
        return carry

    lax.fori_loop(0, SUP, step, 0)


def _nrm2(pos_flat, src, dst):
    kern = pl.kernel(
        _nrm2_body,
        out_type=jax.ShapeDtypeStruct((E,), _f32),
        mesh=_MESH,
        scratch_types=[pltpu.VMEM((3 * N,), _f32),
                       pltpu.VMEM((S,), jnp.int32),
                       pltpu.VMEM((S,), jnp.int32),
                       pltpu.VMEM((S,), _f32)],
        compiler_params=pltpu.CompilerParams(needs_layout_passes=False),
    )
    return kern(pos_flat, src, dst)


CHG = 40
SSG = 5
SG = CHG * SSG
SUPG = PER_W // SG


def _make_pair_combine_body(subtract, n_e):
    per_w = n_e // NW
    supg = per_w // SG

    def body(a_hbm, b_hbm, src_hbm, dst_hbm, gc_hbm,
             is0, is1, id0, id1, ra0, ra1, rb0, rb1,
             sa0, sa1, sb0, sb1, w0, w1):
        wid = lax.axis_index("s") * NC + lax.axis_index("c")
        base = wid * per_w
        isv = [is0, is1]
        idv = [id0, id1]
        ra = [ra0, ra1]
        rb = [rb0, rb1]
        sa = [sa0, sa1]
        sb = [sb0, sb1]
        ws = [w0, w1]

        def fire(t, p):
            off = base + t * SG
            pltpu.sync_copy(src_hbm.at[pl.ds(off, SG)], isv[p])
            pltpu.sync_copy(dst_hbm.at[pl.ds(off, SG)], idv[p])
            for j in range(SSG):
                sl = pl.ds(j * CHG, CHG)
                pltpu.async_copy(a_hbm.at[isv[p].at[sl]], ra[p].at[sl], sa[p])
                pltpu.async_copy(b_hbm.at[idv[p].at[sl]], rb[p].at[sl], sb[p])

        def wait_gathers(p):
            pltpu.make_async_copy(a_hbm.at[pl.ds(0, SG)], ra[p], sa[p]).wait()
            pltpu.make_async_copy(b_hbm.at[pl.ds(0, SG)], rb[p], sb[p]).wait()

        def wait_wb(p):
            pltpu.make_async_copy(a_hbm.at[pl.ds(0, SG)], ra[p], ws[p]).wait()

        def combine_wb(t, p):
            def row(r, carry):
                for c in range(H // 16):
                    sl2 = pl.ds(c * 16, 16)
                    if subtract:
                        ra[p][r, sl2] = ra[p][r, sl2] - rb[p][r, sl2]
                    else:
                        ra[p][r, sl2] = ra[p][r, sl2] + rb[p][r, sl2]
                return carry
            lax.fori_loop(0, SG, row, 0)
            off = base + t * SG
            pltpu.async_copy(ra[p], gc_hbm.at[pl.ds(off, SG)], ws[p])

        fire(0, 0)

        def outer(k, carry):
            for p in range(2):
                t = 2 * k + p
                wait_gathers(p)

                @pl.when(t + 1 < supg)
                def _():
                    @pl.when(t >= 1)
                    def _():
                        wait_wb(1 - p)
                    fire(t + 1, 1 - p)

                combine_wb(t, p)
            return carry

        lax.fori_loop(0, supg // 2, outer, 0)
        if supg % 2 == 1:
            wait_gathers(0)
            combine_wb(supg - 1, 0)
        wait_wb(0)
        wait_wb(1)

    return body


def _pair_combine(a, b, src, dst, subtract):
    n_e = src.shape[0]
    kern = pl.kernel(
        _make_pair_combine_body(subtract, n_e),
        out_type=jax.ShapeDtypeStruct((n_e, H), _f32),
        mesh=_MESH,
        scratch_types=[pltpu.VMEM((SG,), jnp.int32) for _ in range(4)] +
                      [pltpu.VMEM((SG, H), _f32) for _ in range(4)] +
                      [pltpu.SemaphoreType.DMA for _ in range(6)],
    )
    return kern(a, b, src, dst)


_RPT = 624
_TAIL = N - _RPT * NS
S2 = 80
SUP2 = PER_W // S2


def _scatter_body(vals0_hbm, vals1_hbm, dst_hbm, zeros_hbm, out_hbm,
                  i0, i1, v0, v1, acc, s0, s1):
    c = lax.axis_index("c")
    s = lax.axis_index("s")
    idx = [i0, i1]
    val = [v0, v1]
    sem = [s0, s1]
    pltpu.sync_copy(zeros_hbm.at[pl.ds(s * _RPT, _RPT)],
                    acc.at[pl.ds(s * _RPT, _RPT)])

    @pl.when(s == NS - 1)
    def _():
        pltpu.sync_copy(zeros_hbm.at[pl.ds(_RPT * NS, _TAIL)],
                        acc.at[pl.ds(_RPT * NS, _TAIL)])

    plsc.subcore_barrier()

    tile = c * NS + s
    base = tile * PER_W
    lbase = s * PER_W

    def run(vref):
        def load(t, p):
            off = base + t * S2
            pltpu.sync_copy(dst_hbm.at[pl.ds(off, S2)], idx[p])
            pltpu.sync_copy(vref.at[pl.ds(lbase + t * S2, S2)], val[p])

        def drain(p):
            pltpu.make_async_copy(
                vref.at[pl.ds(0, S2)], val[p], sem[p]).wait()

        load(0, 0)

        def outer(k, carry):
            for p in range(2):
                t = 2 * k + p
                pltpu.async_copy(val[p], acc.at[idx[p]], sem[p], add=True)

                @pl.when(t + 1 < SUP2)
                def _():
                    @pl.when(t >= 1)
                    def _():
                        drain(1 - p)
                    load(t + 1, 1 - p)
            return carry

        lax.fori_loop(0, SUP2 // 2, outer, 0)
        pltpu.async_copy(val[0], acc.at[idx[0]], sem[0], add=True)
        drain(1)
        drain(0)

    @pl.when(c == 0)
    def _():
        run(vals0_hbm)

    @pl.when(c == 1)
    def _():
        run(vals1_hbm)

    plsc.subcore_barrier()
    pltpu.sync_copy(acc.at[pl.ds(s * _RPT, _RPT)],
                    out_hbm.at[c, pl.ds(s * _RPT, _RPT)])

    @pl.when(s == NS - 1)
    def _():
        pltpu.sync_copy(acc.at[pl.ds(_RPT * NS, _TAIL)],
                        out_hbm.at[c, pl.ds(_RPT * NS, _TAIL)])


def _scatter_add(vals0, vals1, dst, zeros):
    kern = pl.kernel(
        _scatter_body,
        out_type=jax.ShapeDtypeStruct((NC, N, H), _f32),
        mesh=_MESH,
        scratch_types=[pltpu.VMEM((S2,), jnp.int32) for _ in range(2)] +
                      [pltpu.VMEM((S2, H), _f32) for _ in range(2)] +
                      [pltpu.VMEM_SHARED((N, H), _f32),
                       pltpu.SemaphoreType.DMA, pltpu.SemaphoreType.DMA],
    )
    return kern(vals0, vals1, dst, zeros)


NB = 1000
EB = 4000


def _row_spec(rb, w):
    return pl.BlockSpec((rb, w), lambda i: (i, 0))


def _w_spec(r, c):
    return pl.BlockSpec((r, c), lambda i: (0, 0))


def _node0_body(x_ref, pos_ref, wn0, bn0, wn1, bn1, gn, btn,
                wp0, wx0, wa0, ba0, wb0,
                x1_ref, u_ref, a_ref, b_ref):
    xb = x_ref[...]
    h = _dot(xb, wn0[...]) + bn0[...]
    x1 = _ln(_dot(_elu(h), wn1[...]) + bn1[...], gn[...], btn[...])
    x1_ref[...] = x1
    u_ref[...] = _dot(pos_ref[...], wp0[...]) + _dot(xb, wx0[...])
    a_ref[...] = _dot(x1, wa0[...]) + ba0[...]
    b_ref[...] = _dot(x1, wb0[...])


def _node0(x, pos, wn0, bn0, wn1, bn1, gn, btn, wp0, wx0, wa0, ba0, wb0):
    g = N // NB
    out_shape = tuple(jax.ShapeDtypeStruct((N, H), _f32) for _ in range(4))
    return pl.pallas_call(
        _node0_body,
        grid=(g,),
        in_specs=[_row_spec(NB, H), _row_spec(NB, 3),
                  _w_spec(H, H), _w_spec(1, H), _w_spec(H, H), _w_spec(1, H),
                  _w_spec(1, H), _w_spec(1, H),
                  _w_spec(3, H), _w_spec(H, H), _w_spec(H, H), _w_spec(1, H),
                  _w_spec(H, H)],
        out_specs=tuple(_row_spec(NB, H) for _ in range(4)),
        out_shape=out_shape,
    )(x, pos, wn0, bn0, wn1, bn1, gn, btn, wp0, wx0, wa0, ba0, wb0)


def _enc_upd_edge_body(gd, nrm2, gab, wnr, be0, we1, be1, ge0, bte0,
                       wc0, w1, b1, ge, bte, eo_ref):
    nrm = jnp.sqrt(nrm2[...])
    h0 = gd[...] + nrm * wnr[...] + be0[...]
    e = _ln(_bdot(_elu(h0), we1[...]) + be1[...], ge0[...], bte0[...])
    h = gab[...] + _bdot(e, wc0[...])
    eo_ref[...] = e + _ln(_bdot(_elu(h), w1[...]) + b1[...], ge[...], bte[...])


def _enc_upd_edge(gd, nrm2, gab, wnr, be0, we1, be1, ge0, bte0,
                  wc0, w1, b1, ge, bte):
    n_e = gd.shape[0]
    g = n_e // EB
    return pl.pallas_call(
        _enc_upd_edge_body,
        grid=(g,),
        in_specs=[_row_spec(EB, H), _row_spec(EB, 1), _row_spec(EB, H),
                  _w_spec(1, H), _w_spec(1, H), _w_spec(H, H), _w_spec(1, H),
                  _w_spec(1, H), _w_spec(1, H),
                  _w_spec(H, H), _w_spec(H, H), _w_spec(1, H),
                  _w_spec(1, H), _w_spec(1, H)],
        out_specs=_row_spec(EB, H),
        out_shape=jax.ShapeDtypeStruct((n_e, H), _f32),
    )(gd, nrm2, gab, wnr, be0, we1, be1, ge0, bte0, wc0, w1, b1, ge, bte)


def _upd_edge_body(gab, e, wc0, w1, b1, ge, bte, eo_ref):
    h = gab[...] + _bdot(e[...], wc0[...])
    eo_ref[...] = e[...] + _ln(_bdot(_elu(h), w1[...]) + b1[...],
                               ge[...], bte[...])


def _upd_edge(gab, e, wc0, w1, b1, ge, bte):
    n_e = gab.shape[0]
    g = n_e // EB
    return pl.pallas_call(
        _upd_edge_body,
        grid=(g,),
        in_specs=[_row_spec(EB, H), _row_spec(EB, H),
                  _w_spec(H, H), _w_spec(H, H), _w_spec(1, H),
                  _w_spec(1, H), _w_spec(1, H)],
        out_specs=_row_spec(EB, H),
        out_shape=jax.ShapeDtypeStruct((n_e, H), _f32),
    )(gab, e, wc0, w1, b1, ge, bte)


def _upd_node1_body(x, p0, wn0a, wn0b, b0, w1, b1, gn, btn,
                    wa0, ba0, wb0, xo, ao, bo):
    agg = p0[0] + p0[1]
    h = _dot(x[...], wn0a[...]) + _dot(agg, wn0b[...]) + b0[...]
    xn = x[...] + _ln(_dot(_elu(h), w1[...]) + b1[...], gn[...], btn[...])
    xo[...] = xn
    ao[...] = _dot(xn, wa0[...]) + ba0[...]
    bo[...] = _dot(xn, wb0[...])


def _upd_node1(x, p0, wn0a, wn0b, b0, w1, b1, gn, btn, wa0, ba0, wb0):
    g = N // NB
    parts_spec = pl.BlockSpec((NC, NB, H), lambda i: (0, i, 0))
    return pl.pallas_call(
        _upd_node1_body,
        grid=(g,),
        in_specs=[_row_spec(NB, H), parts_spec,
                  _w_spec(H, H), _w_spec(H, H), _w_spec(1, H),
                  _w_spec(H, H), _w_spec(1, H), _w_spec(1, H), _w_spec(1, H),
                  _w_spec(H, H), _w_spec(1, H), _w_spec(H, H)],
        out_specs=tuple(_row_spec(NB, H) for _ in range(3)),
        out_shape=tuple(jax.ShapeDtypeStruct((N, H), _f32) for _ in range(3)),
    )(x, p0, wn0a, wn0b, b0, w1, b1, gn, btn, wa0, ba0, wb0)


def _upd_node2_body(x, p0, wn0a, wn0b, b0, w1, b1, gn, btn,
                    wd0, bd0, wd1, bd1, out_ref):
    agg = p0[0] + p0[1]
    h = _dot(x[...], wn0a[...]) + _dot(agg, wn0b[...]) + b0[...]
    xn = x[...] + _ln(_dot(_elu(h), w1[...]) + b1[...], gn[...], btn[...])
    hd = _dot(xn, wd0[...]) + bd0[...]
    out_ref[...] = _dot(_elu(hd), wd1[...]) + bd1[...]


def _upd_node2(x, p0, wn0a, wn0b, b0, w1, b1, gn, btn, wd0, bd0, wd1, bd1):
    g = N // NB
    parts_spec = pl.BlockSpec((NC, NB, H), lambda i: (0, i, 0))
    return pl.pallas_call(
        _upd_node2_body,
        grid=(g,),
        in_specs=[_row_spec(NB, H), parts_spec,
                  _w_spec(H, H), _w_spec(H, H), _w_spec(1, H),
                  _w_spec(H, H), _w_spec(1, H), _w_spec(1, H), _w_spec(1, H),
                  _w_spec(H, H), _w_spec(1, H), _w_spec(H, H), _w_spec(1, H)],
        out_specs=_row_spec(NB, H),
        out_shape=jax.ShapeDtypeStruct((N, H), _f32),
    )(x, p0, wn0a, wn0b, b0, w1, b1, gn, btn, wd0, bd0, wd1, bd1)


def _r(v):
    return v.reshape(1, H)


def kernel(x, edge_index, pos, params):
    src = edge_index[0]
    dst = edge_index[1]

    pn, pe, pdec = params["node_enc"], params["edge_enc"], params["dec"]
    mp = params["mp"]

    wn0, wn1 = pn["W"][0].T, pn["W"][1].T
    we0t = pe["W"][0].T
    wp0 = we0t[:3]
    wnr = we0t[3].reshape(1, H)
    wx0 = we0t[4:]
    we1 = pe["W"][1].T

    ew = [lp["edge"]["W"][0].T for lp in mp]
    wa = [w[:H] for w in ew]
    wb = [w[H:2 * H] for w in ew]
    wc = [w[2 * H:] for w in ew]
    ew1 = [lp["edge"]["W"][1].T for lp in mp]
    nw0 = [lp["node"]["W"][0].T for lp in mp]
    wna = [w[:H] for w in nw0]
    wnb = [w[H:] for w in nw0]
    nw1 = [lp["node"]["W"][1].T for lp in mp]
    wd0, wd1 = pdec["W"][0].T, pdec["W"][1].T

    pos_flat = pos.reshape(-1)
    zeros = jnp.zeros((N, H), _f32)

    x1, u, a1, b1 = _node0(
        x, pos, wn0, _r(pn["b"][0]), wn1, _r(pn["b"][1]),
        _r(pn["g"]), _r(pn["beta"]),
        wp0, wx0, wa[0], _r(mp[0]["edge"]["b"][0]), wb[0])

    nrm2 = _nrm2(pos_flat, src, dst).reshape(E, 1)
    EH = E // 2
    srcs = (src[:EH], src[EH:])
    dsts = (dst[:EH], dst[EH:])
    nrm2s = (nrm2[:EH], nrm2[EH:])
    gd = [_pair_combine(u, u, srcs[h], dsts[h], subtract=True)
          for h in range(2)]

    xc = x1
    ab = (a1, b1)
    out = None
    e = [None, None]
    for l, lp in enumerate(mp):
        for h in range(2):
            gab = _pair_combine(ab[0], ab[1], srcs[h], dsts[h],
                                subtract=False)
            if l == 0:
                e[h] = _enc_upd_edge(
                    gd[h], nrm2s[h], gab, wnr, _r(pe["b"][0]),
                    we1, _r(pe["b"][1]), _r(pe["g"]), _r(pe["beta"]),
                    wc[l], ew1[l], _r(lp["edge"]["b"][1]),
                    _r(lp["edge"]["g"]), _r(lp["edge"]["beta"]))
            else:
                e[h] = _upd_edge(gab, e[h], wc[l], ew1[l],
                                 _r(lp["edge"]["b"][1]),
                                 _r(lp["edge"]["g"]), _r(lp["edge"]["beta"]))
        parts = _scatter_add(e[0], e[1], dst, zeros)
        if l + 1 < len(mp):
            xc, a2, b2 = _upd_node1(
                xc, parts, wna[l], wnb[l], _r(lp["node"]["b"][0]),
                nw1[l], _r(lp["node"]["b"][1]),
                _r(lp["node"]["g"]), _r(lp["node"]["beta"]),
                wa[l + 1], _r(mp[l + 1]["edge"]["b"][0]), wb[l + 1])
            ab = (a2, b2)
        else:
            out = _upd_node2(
                xc, parts, wna[l], wnb[l], _r(lp["node"]["b"][0]),
                nw1[l], _r(lp["node"]["b"][1]),
                _r(lp["node"]["g"]), _r(lp["node"]["beta"]),
                wd0, _r(pdec["b"][0]), wd1, _r(pdec["b"][1]))
    return out


# speedup vs baseline: 1.0610x; 1.0610x over previous
import functools

import jax
import jax.numpy as jnp
from jax import lax
from jax.experimental import pallas as pl
from jax.experimental.pallas import tpu as pltpu
from jax.experimental.pallas import tpu_sc as plsc

N = 10000
E = 320000
H = 128

NC, NS = 2, 16
NW = NC * NS
CH = 80
SS = 5
S = CH * SS
PER_W = E // NW
SUP = PER_W // S

_MESH = plsc.VectorSubcoreMesh(
    core_axis_name="c", subcore_axis_name="s", num_cores=NC, num_subcores=NS)

_f32 = jnp.float32
_bf16 = jnp.bfloat16
_dot = functools.partial(jnp.dot, preferred_element_type=jnp.float32)


def _bdot(x, w):
    return jnp.dot(x.astype(_bf16), w.astype(_bf16),
                   preferred_element_type=jnp.float32)


def _elu(h):
    return jnp.where(h > 0, h, jnp.exp(jnp.minimum(h, 0.0)) - 1.0)


def _ln(h, g, beta):
    mu = jnp.mean(h, axis=-1, keepdims=True)
    var = jnp.mean((h - mu) ** 2, axis=-1, keepdims=True)
    return (h - mu) * lax.rsqrt(var + 1e-5) * g + beta


def _nrm2_body(pos_hbm, src_hbm, dst_hbm, nrm2_hbm, pos_v, isv, idv, nv):
    wid = lax.axis_index("s") * NC + lax.axis_index("c")
    base = wid * PER_W
    pltpu.sync_copy(pos_hbm, pos_v)

    def step(t, carry):
        off = base + t * S
        pltpu.sync_copy(src_hbm.at[pl.ds(off, S)], isv)
        pltpu.sync_copy(dst_hbm.at[pl.ds(off, S)], idv)
        for k in range(S // 16):
            sl = pl.ds(k * 16, 16)
            rs = isv[sl] * 3
            rd = idv[sl] * 3
            acc = jnp.zeros((16,), _f32)
            for comp in range(3):
                ps = plsc.load_gather(pos_v, [rs + comp])
                pd_ = plsc.load_gather(pos_v, [rd + comp])
                d = ps - pd_
                acc = acc + d * d
            nv[sl] = acc
        pltpu.sync_copy(nv, nrm2_hbm.at[pl.ds(off, S)])
        return carry

    lax.fori_loop(0, SUP, step, 0)


def _nrm2(pos_flat, src, dst):
    kern = pl.kernel(
        _nrm2_body,
        out_type=jax.ShapeDtypeStruct((E,), _f32),
        mesh=_MESH,
        scratch_types=[pltpu.VMEM((3 * N,), _f32),
                       pltpu.VMEM((S,), jnp.int32),
                       pltpu.VMEM((S,), jnp.int32),
                       pltpu.VMEM((S,), _f32)],
        compiler_params=pltpu.CompilerParams(needs_layout_passes=False),
    )
    return kern(pos_flat, src, dst)


CHG = 40
SSG = 5
SG = CHG * SSG
SUPG = PER_W // SG


def _make_pair_combine_body(subtract, n_e):
    per_w = n_e // NW
    supg = per_w // SG

    def body(a_hbm, b_hbm, src_hbm, dst_hbm, gc_hbm,
             is0, is1, id0, id1, ra0, ra1, rb0, rb1,
             sa0, sa1, sb0, sb1, w0, w1):
        wid = lax.axis_index("s") * NC + lax.axis_index("c")
        base = wid * per_w
        isv = [is0, is1]
        idv = [id0, id1]
        ra = [ra0, ra1]
        rb = [rb0, rb1]
        sa = [sa0, sa1]
        sb = [sb0, sb1]
        ws = [w0, w1]

        def fire(t, p):
            off = base + t * SG
            pltpu.sync_copy(src_hbm.at[pl.ds(off, SG)], isv[p])
            pltpu.sync_copy(dst_hbm.at[pl.ds(off, SG)], idv[p])
            for j in range(SSG):
                sl = pl.ds(j * CHG, CHG)
                pltpu.async_copy(a_hbm.at[isv[p].at[sl]], ra[p].at[sl], sa[p])
                pltpu.async_copy(b_hbm.at[idv[p].at[sl]], rb[p].at[sl], sb[p])

        def wait_gathers(p):
            pltpu.make_async_copy(a_hbm.at[pl.ds(0, SG)], ra[p], sa[p]).wait()
            pltpu.make_async_copy(b_hbm.at[pl.ds(0, SG)], rb[p], sb[p]).wait()

        def wait_wb(p):
            pltpu.make_async_copy(a_hbm.at[pl.ds(0, SG)], ra[p], ws[p]).wait()

        def combine_wb(t, p):
            def row(r, carry):
                for c in range(H // 16):
                    sl2 = pl.ds(c * 16, 16)
                    if subtract:
                        ra[p][r, sl2] = ra[p][r, sl2] - rb[p][r, sl2]
                    else:
                        ra[p][r, sl2] = ra[p][r, sl2] + rb[p][r, sl2]
                return carry
            lax.fori_loop(0, SG, row, 0)
            off = base + t * SG
            pltpu.async_copy(ra[p], gc_hbm.at[pl.ds(off, SG)], ws[p])

        fire(0, 0)

        def outer(k, carry):
            for p in range(2):
                t = 2 * k + p
                wait_gathers(p)

                @pl.when(t + 1 < supg)
                def _():
                    @pl.when(t >= 1)
                    def _():
                        wait_wb(1 - p)
                    fire(t + 1, 1 - p)

                combine_wb(t, p)
            return carry

        lax.fori_loop(0, supg // 2, outer, 0)
        if supg % 2 == 1:
            wait_gathers(0)
            combine_wb(supg - 1, 0)
        wait_wb(0)
        wait_wb(1)

    return body


def _pair_combine(a, b, src, dst, subtract):
    n_e = src.shape[0]
    kern = pl.kernel(
        _make_pair_combine_body(subtract, n_e),
        out_type=jax.ShapeDtypeStruct((n_e, H), _f32),
        mesh=_MESH,
        scratch_types=[pltpu.VMEM((SG,), jnp.int32) for _ in range(4)] +
                      [pltpu.VMEM((SG, H), _f32) for _ in range(4)] +
                      [pltpu.SemaphoreType.DMA for _ in range(6)],
    )
    return kern(a, b, src, dst)


_RPT = 624
_TAIL = N - _RPT * NS
S2 = 80
SUP2 = PER_W // S2


def _scatter_body(vals0_hbm, vals1_hbm, dst_hbm, zeros_hbm, out_hbm,
                  i0, i1, v0, v1, acc, s0, s1):
    c = lax.axis_index("c")
    s = lax.axis_index("s")
    idx = [i0, i1]
    val = [v0, v1]
    sem = [s0, s1]
    pltpu.sync_copy(zeros_hbm.at[pl.ds(s * _RPT, _RPT)],
                    acc.at[pl.ds(s * _RPT, _RPT)])

    @pl.when(s == NS - 1)
    def _():
        pltpu.sync_copy(zeros_hbm.at[pl.ds(_RPT * NS, _TAIL)],
                        acc.at[pl.ds(_RPT * NS, _TAIL)])

    plsc.subcore_barrier()

    tile = c * NS + s
    base = tile * PER_W
    lbase = s * PER_W

    def run(vref):
        def load(t, p):
            off = base + t * S2
            pltpu.sync_copy(dst_hbm.at[pl.ds(off, S2)], idx[p])
            pltpu.sync_copy(vref.at[pl.ds(lbase + t * S2, S2)], val[p])

        def drain(p):
            pltpu.make_async_copy(
                vref.at[pl.ds(0, S2)], val[p], sem[p]).wait()

        load(0, 0)

        def outer(k, carry):
            for p in range(2):
                t = 2 * k + p
                pltpu.async_copy(val[p], acc.at[idx[p]], sem[p], add=True)

                @pl.when(t + 1 < SUP2)
                def _():
                    @pl.when(t >= 1)
                    def _():
                        drain(1 - p)
                    load(t + 1, 1 - p)
            return carry

        lax.fori_loop(0, SUP2 // 2, outer, 0)
        pltpu.async_copy(val[0], acc.at[idx[0]], sem[0], add=True)
        drain(1)
        drain(0)

    @pl.when(c == 0)
    def _():
        run(vals0_hbm)

    @pl.when(c == 1)
    def _():
        run(vals1_hbm)

    plsc.subcore_barrier()
    pltpu.sync_copy(acc.at[pl.ds(s * _RPT, _RPT)],
                    out_hbm.at[c, pl.ds(s * _RPT, _RPT)])

    @pl.when(s == NS - 1)
    def _():
        pltpu.sync_copy(acc.at[pl.ds(_RPT * NS, _TAIL)],
                        out_hbm.at[c, pl.ds(_RPT * NS, _TAIL)])


def _scatter_add(vals0, vals1, dst, zeros):
    kern = pl.kernel(
        _scatter_body,
        out_type=jax.ShapeDtypeStruct((NC, N, H), _f32),
        mesh=_MESH,
        scratch_types=[pltpu.VMEM((S2,), jnp.int32) for _ in range(2)] +
                      [pltpu.VMEM((S2, H), _f32) for _ in range(2)] +
                      [pltpu.VMEM_SHARED((N, H), _f32),
                       pltpu.SemaphoreType.DMA, pltpu.SemaphoreType.DMA],
    )
    return kern(vals0, vals1, dst, zeros)


NB = 1000
EB = 4000


def _row_spec(rb, w):
    return pl.BlockSpec((rb, w), lambda i: (i, 0))


def _w_spec(r, c):
    return pl.BlockSpec((r, c), lambda i: (0, 0))


def _node0_body(x_ref, pos_ref, wn0, bn0, wn1, bn1, gn, btn,
                wp0, wx0, wa0, ba0, wb0,
                x1_ref, u_ref, a_ref, b_ref):
    xb = x_ref[...]
    h = _dot(xb, wn0[...]) + bn0[...]
    x1 = _ln(_dot(_elu(h), wn1[...]) + bn1[...], gn[...], btn[...])
    x1_ref[...] = x1
    u_ref[...] = _dot(pos_ref[...], wp0[...]) + _dot(xb, wx0[...])
    a_ref[...] = _dot(x1, wa0[...]) + ba0[...]
    b_ref[...] = _dot(x1, wb0[...])


def _node0(x, pos, wn0, bn0, wn1, bn1, gn, btn, wp0, wx0, wa0, ba0, wb0):
    g = N // NB
    out_shape = tuple(jax.ShapeDtypeStruct((N, H), _f32) for _ in range(4))
    return pl.pallas_call(
        _node0_body,
        grid=(g,),
        in_specs=[_row_spec(NB, H), _row_spec(NB, 3),
                  _w_spec(H, H), _w_spec(1, H), _w_spec(H, H), _w_spec(1, H),
                  _w_spec(1, H), _w_spec(1, H),
                  _w_spec(3, H), _w_spec(H, H), _w_spec(H, H), _w_spec(1, H),
                  _w_spec(H, H)],
        out_specs=tuple(_row_spec(NB, H) for _ in range(4)),
        out_shape=out_shape,
    )(x, pos, wn0, bn0, wn1, bn1, gn, btn, wp0, wx0, wa0, ba0, wb0)


def _enc_upd_edge_body(gd, nrm2, gab, wnr, be0, we1, be1, ge0, bte0,
                       wc0, w1, b1, ge, bte, eo_ref):
    nrm = jnp.sqrt(nrm2[...].reshape(EB, 1))
    h0 = gd[...] + nrm * wnr[...] + be0[...]
    e = _ln(_bdot(_elu(h0), we1[...]) + be1[...], ge0[...], bte0[...])
    h = gab[...] + _bdot(e, wc0[...])
    eo_ref[...] = e + _ln(_bdot(_elu(h), w1[...]) + b1[...], ge[...], bte[...])


def _enc_upd_edge(gd, nrm2, gab, wnr, be0, we1, be1, ge0, bte0,
                  wc0, w1, b1, ge, bte):
    n_e = gd.shape[0]
    g = n_e // EB
    return pl.pallas_call(
        _enc_upd_edge_body,
        grid=(g,),
        in_specs=[_row_spec(EB, H),
                  pl.BlockSpec((1, 1, EB), lambda i: (i, 0, 0)),
                  _row_spec(EB, H),
                  _w_spec(1, H), _w_spec(1, H), _w_spec(H, H), _w_spec(1, H),
                  _w_spec(1, H), _w_spec(1, H),
                  _w_spec(H, H), _w_spec(H, H), _w_spec(1, H),
                  _w_spec(1, H), _w_spec(1, H)],
        out_specs=_row_spec(EB, H),
        out_shape=jax.ShapeDtypeStruct((n_e, H), _f32),
    )(gd, nrm2, gab, wnr, be0, we1, be1, ge0, bte0, wc0, w1, b1, ge, bte)


def _upd_edge_body(gab, e, wc0, w1, b1, ge, bte, eo_ref):
    h = gab[...] + _bdot(e[...], wc0[...])
    eo_ref[...] = e[...] + _ln(_bdot(_elu(h), w1[...]) + b1[...],
                               ge[...], bte[...])


def _upd_edge(gab, e, wc0, w1, b1, ge, bte):
    n_e = gab.shape[0]
    g = n_e // EB
    return pl.pallas_call(
        _upd_edge_body,
        grid=(g,),
        in_specs=[_row_spec(EB, H), _row_spec(EB, H),
                  _w_spec(H, H), _w_spec(H, H), _w_spec(1, H),
                  _w_spec(1, H), _w_spec(1, H)],
        out_specs=_row_spec(EB, H),
        out_shape=jax.ShapeDtypeStruct((n_e, H), _f32),
    )(gab, e, wc0, w1, b1, ge, bte)


def _upd_node1_body(x, p0, wn0a, wn0b, b0, w1, b1, gn, btn,
                    wa0, ba0, wb0, xo, ao, bo):
    agg = p0[0] + p0[1]
    h = _dot(x[...], wn0a[...]) + _dot(agg, wn0b[...]) + b0[...]
    xn = x[...] + _ln(_dot(_elu(h), w1[...]) + b1[...], gn[...], btn[...])
    xo[...] = xn
    ao[...] = _dot(xn, wa0[...]) + ba0[...]
    bo[...] = _dot(xn, wb0[...])


def _upd_node1(x, p0, wn0a, wn0b, b0, w1, b1, gn, btn, wa0, ba0, wb0):
    g = N // NB
    parts_spec = pl.BlockSpec((NC, NB, H), lambda i: (0, i, 0))
    return pl.pallas_call(
        _upd_node1_body,
        grid=(g,),
        in_specs=[_row_spec(NB, H), parts_spec,
                  _w_spec(H, H), _w_spec(H, H), _w_spec(1, H),
                  _w_spec(H, H), _w_spec(1, H), _w_spec(1, H), _w_spec(1, H),
                  _w_spec(H, H), _w_spec(1, H), _w_spec(H, H)],
        out_specs=tuple(_row_spec(NB, H) for _ in range(3)),
        out_shape=tuple(jax.ShapeDtypeStruct((N, H), _f32) for _ in range(3)),
    )(x, p0, wn0a, wn0b, b0, w1, b1, gn, btn, wa0, ba0, wb0)


def _upd_node2_body(x, p0, wn0a, wn0b, b0, w1, b1, gn, btn,
                    wd0, bd0, wd1, bd1, out_ref):
    agg = p0[0] + p0[1]
    h = _dot(x[...], wn0a[...]) + _dot(agg, wn0b[...]) + b0[...]
    xn = x[...] + _ln(_dot(_elu(h), w1[...]) + b1[...], gn[...], btn[...])
    hd = _dot(xn, wd0[...]) + bd0[...]
    out_ref[...] = _dot(_elu(hd), wd1[...]) + bd1[...]


def _upd_node2(x, p0, wn0a, wn0b, b0, w1, b1, gn, btn, wd0, bd0, wd1, bd1):
    g = N // NB
    parts_spec = pl.BlockSpec((NC, NB, H), lambda i: (0, i, 0))
    return pl.pallas_call(
        _upd_node2_body,
        grid=(g,),
        in_specs=[_row_spec(NB, H), parts_spec,
                  _w_spec(H, H), _w_spec(H, H), _w_spec(1, H),
                  _w_spec(H, H), _w_spec(1, H), _w_spec(1, H), _w_spec(1, H),
                  _w_spec(H, H), _w_spec(1, H), _w_spec(H, H), _w_spec(1, H)],
        out_specs=_row_spec(NB, H),
        out_shape=jax.ShapeDtypeStruct((N, H), _f32),
    )(x, p0, wn0a, wn0b, b0, w1, b1, gn, btn, wd0, bd0, wd1, bd1)


def _r(v):
    return v.reshape(1, H)


def kernel(x, edge_index, pos, params):
    src = edge_index[0]
    dst = edge_index[1]

    pn, pe, pdec = params["node_enc"], params["edge_enc"], params["dec"]
    mp = params["mp"]

    wn0, wn1 = pn["W"][0].T, pn["W"][1].T
    we0t = pe["W"][0].T
    wp0 = we0t[:3]
    wnr = we0t[3].reshape(1, H)
    wx0 = we0t[4:]
    we1 = pe["W"][1].T

    ew = [lp["edge"]["W"][0].T for lp in mp]
    wa = [w[:H] for w in ew]
    wb = [w[H:2 * H] for w in ew]
    wc = [w[2 * H:] for w in ew]
    ew1 = [lp["edge"]["W"][1].T for lp in mp]
    nw0 = [lp["node"]["W"][0].T for lp in mp]
    wna = [w[:H] for w in nw0]
    wnb = [w[H:] for w in nw0]
    nw1 = [lp["node"]["W"][1].T for lp in mp]
    wd0, wd1 = pdec["W"][0].T, pdec["W"][1].T

    pos_flat = pos.reshape(-1)
    zeros = jnp.zeros((N, H), _f32)

    x1, u, a1, b1 = _node0(
        x, pos, wn0, _r(pn["b"][0]), wn1, _r(pn["b"][1]),
        _r(pn["g"]), _r(pn["beta"]),
        wp0, wx0, wa[0], _r(mp[0]["edge"]["b"][0]), wb[0])

    nrm2 = _nrm2(pos_flat, src, dst)
    EH = E // 2
    srcs = (src[:EH], src[EH:])
    dsts = (dst[:EH], dst[EH:])
    nrm2s = (nrm2[:EH], nrm2[EH:])
    gd = [_pair_combine(u, u, srcs[h], dsts[h], subtract=True)
          for h in range(2)]

    xc = x1
    ab = (a1, b1)
    out = None
    e = [None, None]
    for l, lp in enumerate(mp):
        for h in range(2):
            gab = _pair_combine(ab[0], ab[1], srcs[h], dsts[h],
                                subtract=False)
            if l == 0:
                e[h] = _enc_upd_edge(
                    gd[h], nrm2s[h].reshape(EH // EB, 1, EB), gab,
                    wnr, _r(pe["b"][0]),
                    we1, _r(pe["b"][1]), _r(pe["g"]), _r(pe["beta"]),
                    wc[l], ew1[l], _r(lp["edge"]["b"][1]),
                    _r(lp["edge"]["g"]), _r(lp["edge"]["beta"]))
            else:
                e[h] = _upd_edge(gab, e[h], wc[l], ew1[l],
                                 _r(lp["edge"]["b"][1]),
                                 _r(lp["edge"]["g"]), _r(lp["edge"]["beta"]))
        parts = _scatter_add(e[0], e[1], dst, zeros)
        if l + 1 < len(mp):
            xc, a2, b2 = _upd_node1(
                xc, parts, wna[l], wnb[l], _r(lp["node"]["b"][0]),
                nw1[l], _r(lp["node"]["b"][1]),
                _r(lp["node"]["g"]), _r(lp["node"]["beta"]),
                wa[l + 1], _r(mp[l + 1]["edge"]["b"][0]), wb[l + 1])
            ab = (a2, b2)
        else:
            out = _upd_node2(
                xc, parts, wna[l], wnb[l], _r(lp["node"]["b"][0]),
                nw1[l], _r(lp["node"]["b"][1]),
                _r(lp["node"]["g"]), _r(lp["node"]["beta"]),
                wd0, _r(pdec["b"][0]), wd1, _r(pdec["b"][1]))
    return out
